# baseline (speedup 1.0000x reference)
import functools

import jax
import jax.numpy as jnp
from jax import lax
from jax.experimental import pallas as pl
from jax.experimental.pallas import tpu as pltpu
from jax.experimental.pallas import tpu_sc as plsc

D_MODEL = 1024
D_HIDDEN = 4096
N_LATENTS = 64
D_LATENT = 1024
N_HEADS = 16
DH = D_LATENT // N_HEADS
N_EXPERTS = 8
TOP_K = 2
B, T = 2, 2048
N = B * T
TILE = 256
MT = 256
PN = N * TOP_K + N_EXPERTS * MT
NMB = PN // MT
HT = 1024
NH = D_HIDDEN // HT

_f32 = jnp.float32
_HP = lax.Precision.HIGHEST


def _softmax_lanes(s):
    m = jnp.max(s, axis=-1, keepdims=True)
    e = jnp.exp(s - m)
    return e / jnp.sum(e, axis=-1, keepdims=True)


def _ln(xt, g, b):
    mu = jnp.mean(xt, axis=-1, keepdims=True)
    var = jnp.sum((xt - mu) ** 2, axis=-1, keepdims=True) * (1.0 / (D_MODEL - 1))
    return g * (xt - mu) / (jnp.sqrt(var) + 1e-6) + b


def _proj_body(x_ref, ce_ref, ss_ref, wk_ref, wkp_ref, wv_ref, wq_ref, wqp_ref,
               k_ref, v_ref, q_ref):
    xt = x_ref[...]
    ce = ce_ref[...]
    ss = ss_ref[...]
    ak = jnp.dot(xt, wk_ref[...], preferred_element_type=_f32, precision=_HP)
    bk = jnp.dot(xt, wkp_ref[...], preferred_element_type=_f32, precision=_HP)
    k_ref[...] = ak * ce + bk * ss
    v_ref[...] = jnp.dot(xt, wv_ref[...], preferred_element_type=_f32, precision=_HP)
    aq = jnp.dot(xt, wq_ref[...], preferred_element_type=_f32, precision=_HP)
    bq = jnp.dot(xt, wqp_ref[...], preferred_element_type=_f32, precision=_HP)
    q_ref[...] = aq * ce + bq * ss


def _proj(xf, ce, ss, wk, wkp, wv, wq, wqp):
    nt = N // TILE
    row = pl.BlockSpec((TILE, D_MODEL), lambda i: (i, 0))
    tri = pl.BlockSpec((TILE, D_LATENT), lambda i: (i % (T // TILE), 0))
    wsp = pl.BlockSpec((D_MODEL, D_LATENT), lambda i: (0, 0))
    return pl.pallas_call(
        _proj_body,
        grid=(nt,),
        in_specs=[row, tri, tri, wsp, wsp, wsp, wsp, wsp],
        out_specs=[row, row, row],
        out_shape=[jax.ShapeDtypeStruct((N, D_LATENT), _f32)] * 3,
        compiler_params=pltpu.CompilerParams(
            dimension_semantics=("arbitrary",)),
    )(xf, ce, ss, wk, wkp, wv, wq, wqp)


def _latent_body(kp_ref, vp_ref, l_ref, wql_ref, wkl_ref, wvl_ref,
                 kz_ref, vz_ref, z_scr, z2_scr):
    q = jnp.dot(l_ref[...], wql_ref[...], preferred_element_type=_f32, precision=_HP)
    for h in range(N_HEADS):
        sl = slice(h * DH, (h + 1) * DH)
        s = lax.dot_general(q[:, sl], kp_ref[:, sl],
                            (((1,), (1,)), ((), ())),
                            preferred_element_type=_f32, precision=_HP) * (DH ** -0.5)
        p = _softmax_lanes(s)
        z_scr[:, sl] = jnp.dot(p, vp_ref[:, sl], preferred_element_type=_f32, precision=_HP)
    z = z_scr[...]
    ql = jnp.dot(z, wql_ref[...], preferred_element_type=_f32, precision=_HP)
    kl = jnp.dot(z, wkl_ref[...], preferred_element_type=_f32, precision=_HP)
    vl = jnp.dot(z, wvl_ref[...], preferred_element_type=_f32, precision=_HP)
    for h in range(N_HEADS):
        sl = slice(h * DH, (h + 1) * DH)
        s = lax.dot_general(ql[:, sl], kl[:, sl],
                            (((1,), (1,)), ((), ())),
                            preferred_element_type=_f32, precision=_HP) * (DH ** -0.5)
        p = _softmax_lanes(s)
        z2_scr[:, sl] = jnp.dot(p, vl[:, sl], preferred_element_type=_f32, precision=_HP)
    z2 = z2_scr[...]
    kz_ref[...] = jnp.dot(z2, wkl_ref[...], preferred_element_type=_f32, precision=_HP)
    vz_ref[...] = jnp.dot(z2, wvl_ref[...], preferred_element_type=_f32, precision=_HP)


def _latent(kp, vp, l_mat, wql, wkl, wvl):
    big = pl.BlockSpec((T, D_LATENT), lambda b: (b, 0))
    wsp = pl.BlockSpec((D_LATENT, D_LATENT), lambda b: (0, 0))
    lsp = pl.BlockSpec((N_LATENTS, D_LATENT), lambda b: (0, 0))
    out = pl.BlockSpec((N_LATENTS, D_LATENT), lambda b: (b, 0))
    return pl.pallas_call(
        _latent_body,
        grid=(B,),
        in_specs=[big, big, lsp, wsp, wsp, wsp],
        out_specs=[out, out],
        out_shape=[jax.ShapeDtypeStruct((B * N_LATENTS, D_LATENT), _f32)] * 2,
        scratch_shapes=[pltpu.VMEM((N_LATENTS, D_LATENT), _f32),
                        pltpu.VMEM((N_LATENTS, D_LATENT), _f32)],
        compiler_params=pltpu.CompilerParams(
            dimension_semantics=("arbitrary",)),
    )(kp, vp, l_mat, wql, wkl, wvl)


def _attn3_body(qx_ref, x_ref, kz_ref, vz_ref, g1_ref, b1_ref, wout_ref,
                wr_ref, br_ref, x1_ref, route_ref, xl_scr):
    for h in range(N_HEADS):
        sl = slice(h * DH, (h + 1) * DH)
        s = lax.dot_general(qx_ref[:, sl], kz_ref[:, sl],
                            (((1,), (1,)), ((), ())),
                            preferred_element_type=_f32, precision=_HP) * (DH ** -0.5)
        p = _softmax_lanes(s)
        xl_scr[:, sl] = jnp.dot(p, vz_ref[:, sl], preferred_element_type=_f32, precision=_HP)
    y = jnp.dot(xl_scr[...], wout_ref[...], preferred_element_type=_f32, precision=_HP)
    x1 = _ln(x_ref[...], g1_ref[...], b1_ref[...]) + y
    x1_ref[...] = x1
    lg = jnp.dot(x1, wr_ref[...], preferred_element_type=_f32, precision=_HP) + br_ref[...]
    li = lax.broadcasted_iota(jnp.int32, lg.shape, 1)
    m1 = jnp.max(lg, axis=-1, keepdims=True)
    i1 = jnp.min(jnp.where(lg == m1, li, N_EXPERTS), axis=-1, keepdims=True)
    lg2 = jnp.where(li == i1, -1e30, lg)
    m2 = jnp.max(lg2, axis=-1, keepdims=True)
    i2 = jnp.min(jnp.where(lg2 == m2, li, N_EXPERTS), axis=-1, keepdims=True)
    d = jnp.exp(m2 - m1)
    p1 = 1.0 / (1.0 + d)
    p2 = d / (1.0 + d)
    lo = lax.broadcasted_iota(jnp.int32, (qx_ref.shape[0], 128), 1)
    route_ref[...] = jnp.where(
        lo == 0, i1.astype(_f32),
        jnp.where(lo == 1, i2.astype(_f32),
                  jnp.where(lo == 2, p1, jnp.where(lo == 3, p2, 0.0))))


def _attn3(qxp, xf, kz, vz, g1, b1, wout, wr, br):
    nt = T // TILE
    row = pl.BlockSpec((TILE, D_MODEL), lambda b, i: (b * nt + i, 0))
    lat = pl.BlockSpec((N_LATENTS, D_LATENT), lambda b, i: (b, 0))
    vec = pl.BlockSpec((1, D_MODEL), lambda b, i: (0, 0))
    wsp = pl.BlockSpec((D_LATENT, D_MODEL), lambda b, i: (0, 0))
    wrs = pl.BlockSpec((D_MODEL, N_EXPERTS), lambda b, i: (0, 0))
    brs = pl.BlockSpec((1, N_EXPERTS), lambda b, i: (0, 0))
    rsp = pl.BlockSpec((TILE, 128), lambda b, i: (b * nt + i, 0))
    return pl.pallas_call(
        _attn3_body,
        grid=(B, nt),
        in_specs=[row, row, lat, lat, vec, vec, wsp, wrs, brs],
        out_specs=[row, rsp],
        out_shape=[jax.ShapeDtypeStruct((N, D_MODEL), _f32),
                   jax.ShapeDtypeStruct((N, 128), _f32)],
        scratch_shapes=[pltpu.VMEM((TILE, D_LATENT), _f32)],
        compiler_params=pltpu.CompilerParams(
            dimension_semantics=("arbitrary", "arbitrary")),
    )(qxp, xf, kz, vz, g1.reshape(1, -1), b1.reshape(1, -1), wout,
      wr, br.reshape(1, -1))


def _moe1_body(be_ref, xg_ref, we_ref, ve_ref, hid_ref):
    xt = xg_ref[...]
    a = jnp.dot(xt, we_ref[0], preferred_element_type=_f32)
    bb = jnp.dot(xt, ve_ref[0], preferred_element_type=_f32)
    hid_ref[...] = a * (bb * jax.nn.sigmoid(bb))


def _moe1(be, xg, we, ve):
    gs = pltpu.PrefetchScalarGridSpec(
        num_scalar_prefetch=1,
        grid=(NH, NMB),
        in_specs=[
            pl.BlockSpec((MT, D_MODEL), lambda h, m, be: (m, 0)),
            pl.BlockSpec((1, D_MODEL, HT), lambda h, m, be: (be[m], 0, h)),
            pl.BlockSpec((1, D_MODEL, HT), lambda h, m, be: (be[m], 0, h)),
        ],
        out_specs=pl.BlockSpec((MT, HT), lambda h, m, be: (m, h)),
    )
    return pl.pallas_call(
        _moe1_body,
        grid_spec=gs,
        out_shape=jax.ShapeDtypeStruct((PN, D_HIDDEN), _f32),
        compiler_params=pltpu.CompilerParams(
            dimension_semantics=("arbitrary", "arbitrary")),
    )(be, xg, we, ve)


def _moe2_body(be_ref, hid_ref, woe_ref, w_ref, yw_ref):
    yw_ref[...] = jnp.dot(hid_ref[...], woe_ref[0],
                          preferred_element_type=_f32) * w_ref[...]


def _moe2(be, hid, woe, wpad):
    gs = pltpu.PrefetchScalarGridSpec(
        num_scalar_prefetch=1,
        grid=(NMB,),
        in_specs=[
            pl.BlockSpec((MT, D_HIDDEN), lambda m, be: (m, 0)),
            pl.BlockSpec((1, D_HIDDEN, D_MODEL), lambda m, be: (be[m], 0, 0)),
            pl.BlockSpec((MT, 1), lambda m, be: (m, 0)),
        ],
        out_specs=pl.BlockSpec((MT, D_MODEL), lambda m, be: (m, 0)),
    )
    return pl.pallas_call(
        _moe2_body,
        grid_spec=gs,
        out_shape=jax.ShapeDtypeStruct((PN, D_MODEL), _f32),
        compiler_params=pltpu.CompilerParams(
            dimension_semantics=("arbitrary",)),
    )(be, hid, woe, wpad)


def _make_sc_gather(n_rows, d, chunk):
    info = plsc.get_sparse_core_info()
    nw = info.num_cores * info.num_subcores
    per_w = n_rows // nw
    nchunks = per_w // chunk
    mesh = plsc.VectorSubcoreMesh(core_axis_name="c", subcore_axis_name="s")

    @functools.partial(
        pl.kernel, mesh=mesh,
        out_type=jax.ShapeDtypeStruct((n_rows, d), _f32),
        scratch_types=[
            pltpu.VMEM((per_w,), jnp.int32),
            pltpu.VMEM((chunk, d), _f32),
            pltpu.VMEM((chunk, d), _f32),
            pltpu.SemaphoreType.DMA,
            pltpu.SemaphoreType.DMA,
            pltpu.SemaphoreType.DMA,
            pltpu.SemaphoreType.DMA,
        ])
    def k(table_hbm, idx_hbm, out_hbm, idx_v, rows_a, rows_b, ga, gb, sa, sb):
        wid = lax.axis_index("s") * info.num_cores + lax.axis_index("c")
        base = wid * per_w
        pltpu.sync_copy(idx_hbm.at[pl.ds(base, per_w)], idx_v)
        bufs = (rows_a, rows_b)
        gsems = (ga, gb)
        ssems = (sa, sb)
        gathers = [None] * nchunks
        stores = [None] * nchunks

        def gather(c):
            cp = pltpu.make_async_copy(
                table_hbm.at[idx_v.at[pl.ds(c * chunk, chunk)]],
                bufs[c % 2], gsems[c % 2])
            cp.start()
            gathers[c] = cp

        def store(c):
            cp = pltpu.make_async_copy(
                bufs[c % 2], out_hbm.at[pl.ds(base + c * chunk, chunk)],
                ssems[c % 2])
            cp.start()
            stores[c] = cp

        gather(0)
        for c in range(nchunks):
            gathers[c].wait()
            if c + 1 < nchunks:
                if c >= 1:
                    stores[c - 1].wait()
                gather(c + 1)
            store(c)
        stores[nchunks - 2].wait()
        stores[nchunks - 1].wait()

    return k


_sc_gather_dispatch = None
_sc_gather_combine = None


def _get_sc_gathers():
    global _sc_gather_dispatch, _sc_gather_combine
    if _sc_gather_dispatch is None:
        _sc_gather_dispatch = _make_sc_gather(PN, D_MODEL, 40)
        _sc_gather_combine = _make_sc_gather(N * TOP_K, D_MODEL, 32)
    return _sc_gather_dispatch, _sc_gather_combine


def _fin_body(x1_ref, ya_ref, yb_ref, g2_ref, b2_ref, g3_ref, b3_ref,
              wlin_ref, blin_ref, out_ref):
    x2 = _ln(x1_ref[...], g2_ref[...], b2_ref[...]) + ya_ref[...] + yb_ref[...]
    y3 = jnp.dot(x2, wlin_ref[...], preferred_element_type=_f32, precision=_HP) + blin_ref[...]
    out_ref[...] = _ln(x2, g3_ref[...], b3_ref[...]) + y3


def _fin(x1, ys, g2, b2, g3, b3, wlin, blin):
    nt = N // TILE
    row = pl.BlockSpec((TILE, D_MODEL), lambda i: (i, 0))
    rowb = pl.BlockSpec((TILE, D_MODEL), lambda i: (nt + i, 0))
    vec = pl.BlockSpec((1, D_MODEL), lambda i: (0, 0))
    wsp = pl.BlockSpec((D_MODEL, D_MODEL), lambda i: (0, 0))
    return pl.pallas_call(
        _fin_body,
        grid=(nt,),
        in_specs=[row, row, rowb, vec, vec, vec, vec, wsp, vec],
        out_specs=row,
        out_shape=jax.ShapeDtypeStruct((N, D_MODEL), _f32),
        compiler_params=pltpu.CompilerParams(
            dimension_semantics=("arbitrary",)),
    )(x1, ys, ys, g2.reshape(1, -1), b2.reshape(1, -1), g3.reshape(1, -1),
      b3.reshape(1, -1), wlin, blin.reshape(1, -1))


def kernel(x, cos, sin, Wr, br, We, Ve, Woe, L, Wq_lat, Wk_in, Wv_in, Wq_in,
           Wk_lat, Wv_lat, Wout_proj, g1, b1, g2, b2, g3, b3, Wlin, blin):
    xf = x.reshape(N, D_MODEL)

    pidx = jnp.arange(D_LATENT)
    j = (pidx % DH) // 2
    ce = cos[:, j]
    ss = sin[:, j] * jnp.where(pidx % 2 == 0, -1.0, 1.0)
    wkp = Wk_in[:, pidx ^ 1]
    wqp = Wq_in[:, pidx ^ 1]

    kp, vp, qxp = _proj(xf, ce, ss, Wk_in, wkp, Wv_in, Wq_in, wqp)
    kz, vz = _latent(kp, vp, L, Wq_lat, Wk_lat, Wv_lat)
    x1, route = _attn3(qxp, xf, kz, vz, g1, b1, Wout_proj, Wr, br)

    ids = route[:, :2].astype(jnp.int32)
    probs = route[:, 2:4]
    eflat = ids.reshape(-1)
    wflat = probs.reshape(-1)
    oh = (eflat[:, None] == jnp.arange(N_EXPERTS)[None, :]).astype(jnp.int32)
    cum = jnp.cumsum(oh, axis=0)
    rank = jnp.take_along_axis(cum - oh, eflat[:, None], axis=1)[:, 0]
    counts = cum[-1]
    padded = ((counts + MT - 1) // MT) * MT
    pend = jnp.cumsum(padded)
    poff = pend - padded
    pos = poff[eflat] + rank
    tok = jnp.arange(N * TOP_K, dtype=jnp.int32) // TOP_K
    gather_idx = jnp.zeros((PN,), jnp.int32).at[pos].set(tok)
    wpad = jnp.zeros((PN, 1), _f32).at[pos, 0].set(wflat)
    bstart = jnp.arange(NMB) * MT
    be = jnp.minimum(jnp.sum(bstart[:, None] >= pend[None, :], axis=1),
                     N_EXPERTS - 1).astype(jnp.int32)
    posr = pos.reshape(N, TOP_K).astype(jnp.int32)
    gflat = jnp.concatenate([posr[:, 0], posr[:, 1]])

    gd, gc = _get_sc_gathers()
    xg = gd(x1, gather_idx)
    hid = _moe1(be, xg, We, Ve)
    yw = _moe2(be, hid, Woe, wpad)
    ys = gc(yw, gflat)

    out = _fin(x1, ys, g2, b2, g3, b3, Wlin, blin)
    return out.reshape(B, T, D_MODEL)


# speedup vs baseline: 1.4643x; 1.4643x over previous
import functools

import jax
import jax.numpy as jnp
from jax import lax
from jax.experimental import pallas as pl
from jax.experimental.pallas import tpu as pltpu
from jax.experimental.pallas import tpu_sc as plsc

D_MODEL = 1024
D_HIDDEN = 4096
N_LATENTS = 64
D_LATENT = 1024
N_HEADS = 16
DH = D_LATENT // N_HEADS
N_EXPERTS = 8
TOP_K = 2
B, T = 2, 2048
N = B * T
TILE = 256
MT = 256
PN = N * TOP_K + N_EXPERTS * MT
NMB = PN // MT
HT = 1024
NH = D_HIDDEN // HT

_f32 = jnp.float32


def _softmax_lanes(s):
    m = jnp.max(s, axis=-1, keepdims=True)
    e = jnp.exp(s - m)
    return e / jnp.sum(e, axis=-1, keepdims=True)


def _ln(xt, g, b):
    mu = jnp.mean(xt, axis=-1, keepdims=True)
    var = jnp.sum((xt - mu) ** 2, axis=-1, keepdims=True) * (1.0 / (D_MODEL - 1))
    return g * (xt - mu) / (jnp.sqrt(var) + 1e-6) + b


def _proj_body(x_ref, ce_ref, ss_ref, wk_ref, wkp_ref, wv_ref, wq_ref, wqp_ref,
               k_ref, v_ref, q_ref):
    xt = x_ref[...]
    ce = ce_ref[...]
    ss = ss_ref[...]
    ak = jnp.dot(xt, wk_ref[...], preferred_element_type=_f32)
    bk = jnp.dot(xt, wkp_ref[...], preferred_element_type=_f32)
    k_ref[...] = ak * ce + bk * ss
    v_ref[...] = jnp.dot(xt, wv_ref[...], preferred_element_type=_f32)
    aq = jnp.dot(xt, wq_ref[...], preferred_element_type=_f32)
    bq = jnp.dot(xt, wqp_ref[...], preferred_element_type=_f32)
    q_ref[...] = aq * ce + bq * ss


def _proj(xf, ce, ss, wk, wkp, wv, wq, wqp):
    nt = N // TILE
    row = pl.BlockSpec((TILE, D_MODEL), lambda i: (i, 0))
    tri = pl.BlockSpec((TILE, D_LATENT), lambda i: (i % (T // TILE), 0))
    wsp = pl.BlockSpec((D_MODEL, D_LATENT), lambda i: (0, 0))
    return pl.pallas_call(
        _proj_body,
        grid=(nt,),
        in_specs=[row, tri, tri, wsp, wsp, wsp, wsp, wsp],
        out_specs=[row, row, row],
        out_shape=[jax.ShapeDtypeStruct((N, D_LATENT), _f32)] * 3,
        compiler_params=pltpu.CompilerParams(
            dimension_semantics=("arbitrary",)),
    )(xf, ce, ss, wk, wkp, wv, wq, wqp)


def _latent_body(kp_ref, vp_ref, l_ref, wql_ref, wkl_ref, wvl_ref,
                 kz_ref, vz_ref, z_scr, z2_scr):
    q = jnp.dot(l_ref[...], wql_ref[...], preferred_element_type=_f32)
    for h in range(N_HEADS):
        sl = slice(h * DH, (h + 1) * DH)
        s = lax.dot_general(q[:, sl], kp_ref[:, sl],
                            (((1,), (1,)), ((), ())),
                            preferred_element_type=_f32) * (DH ** -0.5)
        p = _softmax_lanes(s)
        z_scr[:, sl] = jnp.dot(p, vp_ref[:, sl], preferred_element_type=_f32)
    z = z_scr[...]
    ql = jnp.dot(z, wql_ref[...], preferred_element_type=_f32)
    kl = jnp.dot(z, wkl_ref[...], preferred_element_type=_f32)
    vl = jnp.dot(z, wvl_ref[...], preferred_element_type=_f32)
    for h in range(N_HEADS):
        sl = slice(h * DH, (h + 1) * DH)
        s = lax.dot_general(ql[:, sl], kl[:, sl],
                            (((1,), (1,)), ((), ())),
                            preferred_element_type=_f32) * (DH ** -0.5)
        p = _softmax_lanes(s)
        z2_scr[:, sl] = jnp.dot(p, vl[:, sl], preferred_element_type=_f32)
    z2 = z2_scr[...]
    kz_ref[...] = jnp.dot(z2, wkl_ref[...], preferred_element_type=_f32)
    vz_ref[...] = jnp.dot(z2, wvl_ref[...], preferred_element_type=_f32)


def _latent(kp, vp, l_mat, wql, wkl, wvl):
    big = pl.BlockSpec((T, D_LATENT), lambda b: (b, 0))
    wsp = pl.BlockSpec((D_LATENT, D_LATENT), lambda b: (0, 0))
    lsp = pl.BlockSpec((N_LATENTS, D_LATENT), lambda b: (0, 0))
    out = pl.BlockSpec((N_LATENTS, D_LATENT), lambda b: (b, 0))
    return pl.pallas_call(
        _latent_body,
        grid=(B,),
        in_specs=[big, big, lsp, wsp, wsp, wsp],
        out_specs=[out, out],
        out_shape=[jax.ShapeDtypeStruct((B * N_LATENTS, D_LATENT), _f32)] * 2,
        scratch_shapes=[pltpu.VMEM((N_LATENTS, D_LATENT), _f32),
                        pltpu.VMEM((N_LATENTS, D_LATENT), _f32)],
        compiler_params=pltpu.CompilerParams(
            dimension_semantics=("arbitrary",)),
    )(kp, vp, l_mat, wql, wkl, wvl)


def _attn3_body(qx_ref, x_ref, kz_ref, vz_ref, g1_ref, b1_ref, wout_ref,
                wr_ref, br_ref, x1_ref, route_ref, xl_scr):
    for h in range(N_HEADS):
        sl = slice(h * DH, (h + 1) * DH)
        s = lax.dot_general(qx_ref[:, sl], kz_ref[:, sl],
                            (((1,), (1,)), ((), ())),
                            preferred_element_type=_f32) * (DH ** -0.5)
        p = _softmax_lanes(s)
        xl_scr[:, sl] = jnp.dot(p, vz_ref[:, sl], preferred_element_type=_f32)
    y = jnp.dot(xl_scr[...], wout_ref[...], preferred_element_type=_f32)
    x1 = _ln(x_ref[...], g1_ref[...], b1_ref[...]) + y
    x1_ref[...] = x1
    lg = jnp.dot(x1, wr_ref[...], preferred_element_type=_f32) + br_ref[...]
    li = lax.broadcasted_iota(jnp.int32, lg.shape, 1)
    m1 = jnp.max(lg, axis=-1, keepdims=True)
    i1 = jnp.min(jnp.where(lg == m1, li, N_EXPERTS), axis=-1, keepdims=True)
    lg2 = jnp.where(li == i1, -1e30, lg)
    m2 = jnp.max(lg2, axis=-1, keepdims=True)
    i2 = jnp.min(jnp.where(lg2 == m2, li, N_EXPERTS), axis=-1, keepdims=True)
    d = jnp.exp(m2 - m1)
    p1 = 1.0 / (1.0 + d)
    p2 = d / (1.0 + d)
    lo = lax.broadcasted_iota(jnp.int32, (qx_ref.shape[0], 128), 1)
    route_ref[...] = jnp.where(
        lo == 0, i1.astype(_f32),
        jnp.where(lo == 1, i2.astype(_f32),
                  jnp.where(lo == 2, p1, jnp.where(lo == 3, p2, 0.0))))


def _attn3(qxp, xf, kz, vz, g1, b1, wout, wr, br):
    nt = T // TILE
    row = pl.BlockSpec((TILE, D_MODEL), lambda b, i: (b * nt + i, 0))
    lat = pl.BlockSpec((N_LATENTS, D_LATENT), lambda b, i: (b, 0))
    vec = pl.BlockSpec((1, D_MODEL), lambda b, i: (0, 0))
    wsp = pl.BlockSpec((D_LATENT, D_MODEL), lambda b, i: (0, 0))
    wrs = pl.BlockSpec((D_MODEL, N_EXPERTS), lambda b, i: (0, 0))
    brs = pl.BlockSpec((1, N_EXPERTS), lambda b, i: (0, 0))
    rsp = pl.BlockSpec((TILE, 128), lambda b, i: (b * nt + i, 0))
    return pl.pallas_call(
        _attn3_body,
        grid=(B, nt),
        in_specs=[row, row, lat, lat, vec, vec, wsp, wrs, brs],
        out_specs=[row, rsp],
        out_shape=[jax.ShapeDtypeStruct((N, D_MODEL), _f32),
                   jax.ShapeDtypeStruct((N, 128), _f32)],
        scratch_shapes=[pltpu.VMEM((TILE, D_LATENT), _f32)],
        compiler_params=pltpu.CompilerParams(
            dimension_semantics=("arbitrary", "arbitrary")),
    )(qxp, xf, kz, vz, g1.reshape(1, -1), b1.reshape(1, -1), wout,
      wr, br.reshape(1, -1))


def _moe1_body(be_ref, xg_ref, we_ref, ve_ref, hid_ref):
    xt = xg_ref[...]
    a = jnp.dot(xt, we_ref[0], preferred_element_type=_f32)
    bb = jnp.dot(xt, ve_ref[0], preferred_element_type=_f32)
    hid_ref[...] = a * (bb * jax.nn.sigmoid(bb))


def _moe1(be, xg, we, ve):
    gs = pltpu.PrefetchScalarGridSpec(
        num_scalar_prefetch=1,
        grid=(NH, NMB),
        in_specs=[
            pl.BlockSpec((MT, D_MODEL), lambda h, m, be: (m, 0)),
            pl.BlockSpec((1, D_MODEL, HT), lambda h, m, be: (be[m], 0, h)),
            pl.BlockSpec((1, D_MODEL, HT), lambda h, m, be: (be[m], 0, h)),
        ],
        out_specs=pl.BlockSpec((MT, HT), lambda h, m, be: (m, h)),
    )
    return pl.pallas_call(
        _moe1_body,
        grid_spec=gs,
        out_shape=jax.ShapeDtypeStruct((PN, D_HIDDEN), _f32),
        compiler_params=pltpu.CompilerParams(
            dimension_semantics=("arbitrary", "arbitrary")),
    )(be, xg, we, ve)


def _moe2_body(be_ref, hid_ref, woe_ref, w_ref, yw_ref):
    yw_ref[...] = jnp.dot(hid_ref[...], woe_ref[0],
                          preferred_element_type=_f32) * w_ref[...]


def _moe2(be, hid, woe, wpad):
    gs = pltpu.PrefetchScalarGridSpec(
        num_scalar_prefetch=1,
        grid=(NMB,),
        in_specs=[
            pl.BlockSpec((MT, D_HIDDEN), lambda m, be: (m, 0)),
            pl.BlockSpec((1, D_HIDDEN, D_MODEL), lambda m, be: (be[m], 0, 0)),
            pl.BlockSpec((MT, 1), lambda m, be: (m, 0)),
        ],
        out_specs=pl.BlockSpec((MT, D_MODEL), lambda m, be: (m, 0)),
    )
    return pl.pallas_call(
        _moe2_body,
        grid_spec=gs,
        out_shape=jax.ShapeDtypeStruct((PN, D_MODEL), _f32),
        compiler_params=pltpu.CompilerParams(
            dimension_semantics=("arbitrary",)),
    )(be, hid, woe, wpad)


def _make_sc_gather(n_rows, d, chunk):
    info = plsc.get_sparse_core_info()
    nw = info.num_cores * info.num_subcores
    per_w = n_rows // nw
    nchunks = per_w // chunk
    mesh = plsc.VectorSubcoreMesh(core_axis_name="c", subcore_axis_name="s")

    @functools.partial(
        pl.kernel, mesh=mesh,
        out_type=jax.ShapeDtypeStruct((n_rows, d), _f32),
        scratch_types=[
            pltpu.VMEM((per_w,), jnp.int32),
            pltpu.VMEM((chunk, d), _f32),
            pltpu.VMEM((chunk, d), _f32),
            pltpu.SemaphoreType.DMA,
            pltpu.SemaphoreType.DMA,
            pltpu.SemaphoreType.DMA,
            pltpu.SemaphoreType.DMA,
        ])
    def k(table_hbm, idx_hbm, out_hbm, idx_v, rows_a, rows_b, ga, gb, sa, sb):
        wid = lax.axis_index("s") * info.num_cores + lax.axis_index("c")
        base = wid * per_w
        pltpu.sync_copy(idx_hbm.at[pl.ds(base, per_w)], idx_v)
        bufs = (rows_a, rows_b)
        gsems = (ga, gb)
        ssems = (sa, sb)
        gathers = [None] * nchunks
        stores = [None] * nchunks

        def gather(c):
            cp = pltpu.make_async_copy(
                table_hbm.at[idx_v.at[pl.ds(c * chunk, chunk)]],
                bufs[c % 2], gsems[c % 2])
            cp.start()
            gathers[c] = cp

        def store(c):
            cp = pltpu.make_async_copy(
                bufs[c % 2], out_hbm.at[pl.ds(base + c * chunk, chunk)],
                ssems[c % 2])
            cp.start()
            stores[c] = cp

        gather(0)
        for c in range(nchunks):
            gathers[c].wait()
            if c + 1 < nchunks:
                if c >= 1:
                    stores[c - 1].wait()
                gather(c + 1)
            store(c)
        stores[nchunks - 2].wait()
        stores[nchunks - 1].wait()

    return k


_sc_gather_dispatch = None
_sc_gather_combine = None


def _get_sc_gathers():
    global _sc_gather_dispatch, _sc_gather_combine
    if _sc_gather_dispatch is None:
        _sc_gather_dispatch = _make_sc_gather(PN, D_MODEL, 40)
        _sc_gather_combine = _make_sc_gather(N * TOP_K, D_MODEL, 32)
    return _sc_gather_dispatch, _sc_gather_combine


def _fin_body(x1_ref, ya_ref, yb_ref, g2_ref, b2_ref, g3_ref, b3_ref,
              wlin_ref, blin_ref, out_ref):
    x2 = _ln(x1_ref[...], g2_ref[...], b2_ref[...]) + ya_ref[...] + yb_ref[...]
    y3 = jnp.dot(x2, wlin_ref[...], preferred_element_type=_f32) + blin_ref[...]
    out_ref[...] = _ln(x2, g3_ref[...], b3_ref[...]) + y3


def _fin(x1, ys, g2, b2, g3, b3, wlin, blin):
    nt = N // TILE
    row = pl.BlockSpec((TILE, D_MODEL), lambda i: (i, 0))
    rowb = pl.BlockSpec((TILE, D_MODEL), lambda i: (nt + i, 0))
    vec = pl.BlockSpec((1, D_MODEL), lambda i: (0, 0))
    wsp = pl.BlockSpec((D_MODEL, D_MODEL), lambda i: (0, 0))
    return pl.pallas_call(
        _fin_body,
        grid=(nt,),
        in_specs=[row, row, rowb, vec, vec, vec, vec, wsp, vec],
        out_specs=row,
        out_shape=jax.ShapeDtypeStruct((N, D_MODEL), _f32),
        compiler_params=pltpu.CompilerParams(
            dimension_semantics=("arbitrary",)),
    )(x1, ys, ys, g2.reshape(1, -1), b2.reshape(1, -1), g3.reshape(1, -1),
      b3.reshape(1, -1), wlin, blin.reshape(1, -1))


def kernel(x, cos, sin, Wr, br, We, Ve, Woe, L, Wq_lat, Wk_in, Wv_in, Wq_in,
           Wk_lat, Wv_lat, Wout_proj, g1, b1, g2, b2, g3, b3, Wlin, blin):
    xf = x.reshape(N, D_MODEL)

    pidx = jnp.arange(D_LATENT)
    j = (pidx % DH) // 2
    ce = cos[:, j]
    ss = sin[:, j] * jnp.where(pidx % 2 == 0, -1.0, 1.0)
    wkp = Wk_in[:, pidx ^ 1]
    wqp = Wq_in[:, pidx ^ 1]

    kp, vp, qxp = _proj(xf, ce, ss, Wk_in, wkp, Wv_in, Wq_in, wqp)
    kz, vz = _latent(kp, vp, L, Wq_lat, Wk_lat, Wv_lat)
    x1, route = _attn3(qxp, xf, kz, vz, g1, b1, Wout_proj, Wr, br)

    ids = route[:, :2].astype(jnp.int32)
    probs = route[:, 2:4]
    eflat = ids.reshape(-1)
    wflat = probs.reshape(-1)
    oh = (eflat[:, None] == jnp.arange(N_EXPERTS)[None, :]).astype(jnp.int32)
    cum = jnp.cumsum(oh, axis=0)
    rank = jnp.take_along_axis(cum - oh, eflat[:, None], axis=1)[:, 0]
    counts = cum[-1]
    padded = ((counts + MT - 1) // MT) * MT
    pend = jnp.cumsum(padded)
    poff = pend - padded
    pos = poff[eflat] + rank
    tok = jnp.arange(N * TOP_K, dtype=jnp.int32) // TOP_K
    gather_idx = jnp.zeros((PN,), jnp.int32).at[pos].set(tok)
    wpad = jnp.zeros((PN, 1), _f32).at[pos, 0].set(wflat)
    bstart = jnp.arange(NMB) * MT
    be = jnp.minimum(jnp.sum(bstart[:, None] >= pend[None, :], axis=1),
                     N_EXPERTS - 1).astype(jnp.int32)
    posr = pos.reshape(N, TOP_K).astype(jnp.int32)
    gflat = jnp.concatenate([posr[:, 0], posr[:, 1]])

    gd, gc = _get_sc_gathers()
    xg = gd(x1, gather_idx)
    hid = _moe1(be, xg, We, Ve)
    yw = _moe2(be, hid, Woe, wpad)
    ys = gc(yw, gflat)

    out = _fin(x1, ys, g2, b2, g3, b3, Wlin, blin)
    return out.reshape(B, T, D_MODEL)
